# baseline (speedup 1.0000x reference)
import functools

import jax
import jax.numpy as jnp
from jax import lax
from jax.experimental import pallas as pl
from jax.experimental.pallas import tpu as pltpu
from jax.experimental.pallas import tpu_sc as plsc

NC = 2
NS = 16
NW = NC * NS
L = 16

K = 16
D = 128
CHUNK = 8
ROWS = CHUNK * K


def _sc_gather_mean(b_pad):
    npw = b_pad // NW
    nch = npw // CHUNK

    mesh = plsc.VectorSubcoreMesh(core_axis_name="c", subcore_axis_name="s")

    @functools.partial(
        pl.kernel,
        mesh=mesh,
        out_type=jax.ShapeDtypeStruct((b_pad, D), jnp.float32),
        scratch_types=[
            pltpu.VMEM((npw * K,), jnp.int32),
            pltpu.VMEM((ROWS, D), jnp.float32),
            pltpu.VMEM((CHUNK, D), jnp.float32),
            pltpu.SemaphoreType.DMA,
        ],
    )
    def body(feat_hbm, idx_hbm, out_hbm, idx_v, rows_v, acc_v, gsem):
        wid = lax.axis_index("s") * NC + lax.axis_index("c")
        node_base = wid * npw
        pltpu.sync_copy(idx_hbm.at[pl.ds(node_base * K, npw * K)], idx_v)

        def chunk_body(c, carry):
            idx_slice = idx_v.at[pl.ds(c * ROWS, ROWS)]
            pltpu.async_copy(feat_hbm.at[idx_slice], rows_v, gsem).wait()
            for n in range(CHUNK):
                for v in range(D // L):
                    s = rows_v[n * K, pl.ds(v * L, L)]
                    for j in range(1, K):
                        s = s + rows_v[n * K + j, pl.ds(v * L, L)]
                    acc_v[n, pl.ds(v * L, L)] = s * (1.0 / K)
            pltpu.sync_copy(acc_v,
                            out_hbm.at[pl.ds(node_base + c * CHUNK, CHUNK), :])
            return carry

        lax.fori_loop(0, nch, chunk_body, 0)

    return body


def _tc_matmul(b_pad, p, bm):
    def mm_body(x_ref, w_ref, o_ref):
        o_ref[...] = lax.dot_general(
            x_ref[...], w_ref[...], (((1,), (1,)), ((), ())),
            preferred_element_type=jnp.float32)

    return pl.pallas_call(
        mm_body,
        grid=(b_pad // bm,),
        in_specs=[
            pl.BlockSpec((bm, D), lambda i: (i, 0)),
            pl.BlockSpec((p, D), lambda i: (0, 0)),
        ],
        out_specs=pl.BlockSpec((bm, p), lambda i: (i, 0)),
        out_shape=jax.ShapeDtypeStruct((b_pad, p), jnp.float32),
    )


def kernel(features, neigh_idx, W):
    b, k = neigh_idx.shape
    u, d = features.shape
    p = W.shape[0]
    assert k == K and d == D

    step = NW * CHUNK
    b_pad = ((b + step - 1) // step) * step

    idx = neigh_idx.astype(jnp.int32).reshape(-1)
    if b_pad != b:
        idx = jnp.pad(idx, (0, (b_pad - b) * K))

    pooled = _sc_gather_mean(b_pad)(features, idx)
    out = _tc_matmul(b_pad, p, 1024)(pooled, W)
    return out[:b]


# speedup vs baseline: 1.2070x; 1.2070x over previous
import functools

import jax
import jax.numpy as jnp
from jax import lax
from jax.experimental import pallas as pl
from jax.experimental.pallas import tpu as pltpu
from jax.experimental.pallas import tpu_sc as plsc

NC = 2
NS = 16
NW = NC * NS
L = 16

K = 16
D = 128
CHUNK = 8
ROWS = CHUNK * K


def _tree_sum(terms):
    while len(terms) > 1:
        terms = [terms[i] + terms[i + 1] for i in range(0, len(terms), 2)]
    return terms[0]


def _sc_gather_mean(b_pad):
    npw = b_pad // NW
    nch = npw // CHUNK
    assert nch % 2 == 0

    mesh = plsc.VectorSubcoreMesh(core_axis_name="c", subcore_axis_name="s")

    @functools.partial(
        pl.kernel,
        mesh=mesh,
        out_type=jax.ShapeDtypeStruct((b_pad, D), jnp.float32),
        scratch_types=[
            pltpu.VMEM((npw * K,), jnp.int32),
            pltpu.VMEM((2, ROWS, D), jnp.float32),
            pltpu.VMEM((2, CHUNK, D), jnp.float32),
            pltpu.SemaphoreType.DMA,
            pltpu.SemaphoreType.DMA,
            pltpu.SemaphoreType.DMA,
            pltpu.SemaphoreType.DMA,
        ],
    )
    def body(feat_hbm, idx_hbm, out_hbm, idx_v, rows_v, acc_v,
             gsem0, gsem1, osem0, osem1):
        gsems = (gsem0, gsem1)
        osems = (osem0, osem1)
        wid = lax.axis_index("s") * NC + lax.axis_index("c")
        node_base = wid * npw
        pltpu.sync_copy(idx_hbm.at[pl.ds(node_base * K, npw * K)], idx_v)

        def start_gather(chunk, buf):
            idx_slice = idx_v.at[pl.ds(chunk * ROWS, ROWS)]
            pltpu.async_copy(feat_hbm.at[idx_slice], rows_v.at[buf], gsems[buf])

        def wait_gather(buf):
            pltpu.make_async_copy(
                feat_hbm.at[idx_v.at[pl.ds(0, ROWS)]],
                rows_v.at[buf], gsems[buf]).wait()

        def wait_out(buf):
            pltpu.make_async_copy(
                acc_v.at[buf], out_hbm.at[pl.ds(0, CHUNK), :],
                osems[buf]).wait()

        def compute(buf, chunk):
            for n in range(CHUNK):
                for v in range(D // L):
                    s = _tree_sum([rows_v[buf, n * K + j, pl.ds(v * L, L)]
                                   for j in range(K)])
                    acc_v[buf, n, pl.ds(v * L, L)] = s * (1.0 / K)
            pltpu.async_copy(
                acc_v.at[buf],
                out_hbm.at[pl.ds(node_base + chunk * CHUNK, CHUNK), :],
                osems[buf])

        start_gather(0, 0)

        def pair_body(i, carry):
            for b in range(2):
                chunk = 2 * i + b

                @pl.when(chunk + 1 < nch)
                def _():
                    start_gather(chunk + 1, 1 - b)

                wait_gather(b)

                @pl.when(chunk >= 2)
                def _():
                    wait_out(b)

                compute(b, chunk)
            return carry

        lax.fori_loop(0, nch // 2, pair_body, 0)
        wait_out(0)
        wait_out(1)

    return body


def _tc_matmul(b_pad, p, bm):
    def mm_body(x_ref, w_ref, o_ref):
        o_ref[...] = lax.dot_general(
            x_ref[...], w_ref[...], (((1,), (1,)), ((), ())),
            preferred_element_type=jnp.float32)

    return pl.pallas_call(
        mm_body,
        grid=(b_pad // bm,),
        in_specs=[
            pl.BlockSpec((bm, D), lambda i: (i, 0)),
            pl.BlockSpec((p, D), lambda i: (0, 0)),
        ],
        out_specs=pl.BlockSpec((bm, p), lambda i: (i, 0)),
        out_shape=jax.ShapeDtypeStruct((b_pad, p), jnp.float32),
    )


def kernel(features, neigh_idx, W):
    b, k = neigh_idx.shape
    u, d = features.shape
    p = W.shape[0]
    assert k == K and d == D

    step = NW * CHUNK
    b_pad = ((b + step - 1) // step) * step

    idx = neigh_idx.astype(jnp.int32).reshape(-1)
    if b_pad != b:
        idx = jnp.pad(idx, (0, (b_pad - b) * K))

    pooled = _sc_gather_mean(b_pad)(features, idx)
    out = _tc_matmul(b_pad, p, 1024)(pooled, W)
    return out[:b]
